# baseline (speedup 1.0000x reference)
import jax
import jax.numpy as jnp
from jax import lax
from jax.experimental import pallas as pl
from jax.experimental.pallas import tpu as pltpu

_B, _H, _W, _C = 16, 32, 32, 384
_K = int(_H * _W * 0.2)
_SUPPRESS = 0.1


def _shift2d_b(a, dr, dc, pad):
    B, H, W = a.shape
    if dr > 0:
        a = jnp.concatenate([jnp.full((B, dr, W), pad, a.dtype), a[:, :-dr, :]], axis=1)
    elif dr < 0:
        a = jnp.concatenate([a[:, -dr:, :], jnp.full((B, -dr, W), pad, a.dtype)], axis=1)
    if dc > 0:
        a = jnp.concatenate([jnp.full((B, H, dc), pad, a.dtype), a[:, :, :-dc]], axis=2)
    elif dc < 0:
        a = jnp.concatenate([a[:, :, -dc:], jnp.full((B, H, -dc), pad, a.dtype)], axis=2)
    return a


def _body(x_ref, o_ref, xs_ref, s_ref, s8_ref, mask_ref):
    i = pl.program_id(0)

    @pl.when(i < _B)
    def _phase1():
        x = x_ref[0]
        xs_ref[pl.ds(i, 1)] = x_ref[...]
        s = jnp.sum(jnp.abs(x), axis=2)
        s_ref[pl.ds(i, 1)] = s[None]
        s8_ref[pl.ds(i, 1)] = s.reshape(8, 128)[None]

    @pl.when(i == _B)
    def _phase2():
        si8 = lax.bitcast_convert_type(s8_ref[...], jnp.int32)
        lo = jnp.zeros((_B, 1, 1), jnp.int32)
        hi = jnp.full((_B, 1, 1), 0x7FFFFFFF, jnp.int32)
        for _ in range(31):
            mid = lo + ((hi - lo) >> 1)
            cnt = jnp.sum((si8 >= mid).astype(jnp.int32), axis=(1, 2), keepdims=True)
            ge = cnt >= _K
            lo = jnp.where(ge, mid, lo)
            hi = jnp.where(ge, hi, mid)
        si = lax.bitcast_convert_type(s_ref[...], jnp.int32)
        m = si
        for dr in (-1, 0, 1):
            for dc in (-1, 0, 1):
                if dr == 0 and dc == 0:
                    continue
                m = jnp.maximum(m, _shift2d_b(si, dr, dc, jnp.int32(-1)))
        mask_ref[...] = jnp.where(m >= lo, jnp.float32(_SUPPRESS), jnp.float32(1.0))

    @pl.when(i >= _B)
    def _phase3():
        b = i - _B
        o_ref[0] = xs_ref[b] * mask_ref[b][:, :, None]


@jax.jit
def kernel(x):
    return pl.pallas_call(
        _body,
        grid=(2 * _B,),
        in_specs=[
            pl.BlockSpec(
                (1, _H, _W, _C),
                lambda i: (jnp.minimum(i, _B - 1), 0, 0, 0),
            )
        ],
        out_specs=pl.BlockSpec(
            (1, _H, _W, _C),
            lambda i: (jnp.maximum(i - _B, 0), 0, 0, 0),
        ),
        out_shape=jax.ShapeDtypeStruct((_B, _H, _W, _C), jnp.float32),
        scratch_shapes=[
            pltpu.VMEM((_B, _H, _W, _C), jnp.float32),
            pltpu.VMEM((_B, _H, _W), jnp.float32),
            pltpu.VMEM((_B, 8, 128), jnp.float32),
            pltpu.VMEM((_B, _H, _W), jnp.float32),
        ],
    )(x)
